# baseline (speedup 1.0000x reference)
import functools

import jax
import jax.numpy as jnp
from jax import lax
from jax.experimental import pallas as pl
from jax.experimental.pallas import tpu as pltpu
from jax.experimental.pallas import tpu_sc as plsc

H = 32
HH = 16
NC = 2
NS = 16
K = 128
BLK1 = 4
BLKS = 8
NBUF = 4

LN_EPS = 1e-5


def _mlp_ln(x, w1_ref, b1_ref, w2_ref, b2_ref, g_ref, bb_ref):
    h = jnp.dot(x, w1_ref[...], preferred_element_type=jnp.float32) + b1_ref[...]
    h = jnp.dot(h, w2_ref[...], preferred_element_type=jnp.float32) + b2_ref[...]
    m = jnp.mean(h, axis=-1, keepdims=True)
    v = jnp.mean((h - m) ** 2, axis=-1, keepdims=True)
    return (h - m) * lax.rsqrt(v + LN_EPS) * g_ref[...] + bb_ref[...]


def _edge_body(x_ref, w1_ref, b1_ref, w2_ref, b2_ref, g_ref, bb_ref, out_ref):
    y = _mlp_ln(x_ref[...], w1_ref, b1_ref, w2_ref, b2_ref, g_ref, bb_ref)
    out_ref[0, :, :] = y[:, :HH]
    out_ref[1, :, :] = y[:, HH:]


def _node_init_body(x_ref, w1_ref, b1_ref, w2_ref, b2_ref, g_ref, bb_ref,
                    wes_ref, wed_ref,
                    hn_ref, a_ref, t_ref, b_ref):
    hn = _mlp_ln(x_ref[...], w1_ref, b1_ref, w2_ref, b2_ref, g_ref, bb_ref)
    a = jnp.dot(hn, wes_ref[...], preferred_element_type=jnp.float32)
    b = jnp.dot(hn, wed_ref[...], preferred_element_type=jnp.float32)
    hn_ref[...] = hn
    a_ref[...] = a
    t_ref[0, :, :] = a[:, :HH]
    t_ref[1, :, :] = a[:, HH:]
    b_ref[...] = b


def _node_update(hn_ref, g_ref, s0_ref, deg_ref, bt_ref, r_ref, dl_ref,
                 wnh_ref, wna_ref, bn_ref):
    hn = hn_ref[...]
    g = jnp.concatenate([g_ref[0], g_ref[1]], axis=-1)
    s0 = jnp.concatenate([s0_ref[0], s0_ref[1]], axis=-1)
    dg = deg_ref[...]
    aggsum = (jnp.dot(s0, r_ref[...], preferred_element_type=jnp.float32)
              + g + dg * bt_ref[...] + dg * dl_ref[...][None, :])
    agg = aggsum / jnp.maximum(dg, 1.0)
    return (hn + jnp.dot(hn, wnh_ref[...], preferred_element_type=jnp.float32)
            + jnp.dot(agg, wna_ref[...], preferred_element_type=jnp.float32)
            + bn_ref[...])


def _mid_body(hn_ref, g_ref, s0_ref, deg_ref, bt_ref, u_ref, v_ref,
              r_ref, dl_ref, wnh_ref, wna_ref, bn_ref,
              wes_ref, wed_ref, wee_ref,
              hno_ref, to_ref, bto_ref, uo_ref, vo_ref):
    hn2 = _node_update(hn_ref, g_ref, s0_ref, deg_ref, bt_ref, r_ref, dl_ref,
                       wnh_ref, wna_ref, bn_ref)
    a = jnp.dot(hn2, wes_ref[...], preferred_element_type=jnp.float32)
    b = jnp.dot(hn2, wed_ref[...], preferred_element_type=jnp.float32)
    at = a + jnp.dot(u_ref[...], wee_ref[...], preferred_element_type=jnp.float32)
    btn = b + jnp.dot(v_ref[...], wee_ref[...], preferred_element_type=jnp.float32)
    hno_ref[...] = hn2
    to_ref[0, :, :] = at[:, :HH]
    to_ref[1, :, :] = at[:, HH:]
    bto_ref[...] = btn
    uo_ref[...] = u_ref[...] + at
    vo_ref[...] = v_ref[...] + btn


def _final_body(hn_ref, g_ref, s0_ref, deg_ref, bt_ref,
                r_ref, dl_ref, wnh_ref, wna_ref, bn_ref,
                wo1_ref, bo1_ref, wo2_ref, bo2_ref, out_ref):
    hn2 = _node_update(hn_ref, g_ref, s0_ref, deg_ref, bt_ref, r_ref, dl_ref,
                       wnh_ref, wna_ref, bn_ref)
    y = jnp.dot(hn2, wo1_ref[...], preferred_element_type=jnp.float32) + bo1_ref[...]
    out_ref[...] = jnp.dot(y, wo2_ref[...], preferred_element_type=jnp.float32) + bo2_ref[...]


def _full(shape):
    nd = len(shape)
    return pl.BlockSpec(shape, lambda i: (0,) * nd)


_pcall = pl.pallas_call


def _sc_first_body(he_ref, tab_ref, src_ref, dst_ref, dstq_ref, dstr_ref,
                   eye_ref, z16_ref,
                   s0_out, deg_out, g_out,
                   s0acc, qacc, gacc, sidx, didx, qidx, ridx, rows, hrows,
                   erows, gsem0, gsem1, ssem0, ssem1, ssem2, ssem3,
                   hsem0, hsem1, s0sem0, s0sem1, esem0, esem1, qsem0, qsem1,
                   *, n_chunks, zps, npad, qps):
    c = lax.axis_index("c")
    s = lax.axis_index("s")
    wid = c * NS + s
    base = s * zps
    qbase = s * qps
    gsems = (gsem0, gsem1)
    ssems = (ssem0, ssem1, ssem2, ssem3)
    hsems = (hsem0, hsem1)
    s0sems = (s0sem0, s0sem1)
    esems = (esem0, esem1)
    qsems = (qsem0, qsem1)
    pltpu.sync_copy(z16_ref.at[pl.ds(base, zps)], s0acc.at[pl.ds(base, zps)])
    pltpu.sync_copy(z16_ref.at[pl.ds(base, zps)], gacc.at[pl.ds(base, zps)])
    pltpu.sync_copy(z16_ref.at[pl.ds(qbase, qps)], qacc.at[pl.ds(qbase, qps)])
    plsc.subcore_barrier()

    def block(blk, carry):
        pltpu.sync_copy(src_ref.at[wid, pl.ds(blk * BLK1, BLK1)], sidx)
        pltpu.sync_copy(dst_ref.at[s, pl.ds(blk * BLK1, BLK1)], didx)
        pltpu.sync_copy(dstq_ref.at[s, pl.ds(blk * BLK1, BLK1)], qidx)
        pltpu.sync_copy(dstr_ref.at[s, pl.ds(blk * BLK1, BLK1)], ridx)
        hoff = c * (NS * n_chunks * K) + s * (n_chunks * K) + blk * (BLK1 * K)
        g = {0: pltpu.async_copy(tab_ref.at[sidx.at[0]], rows.at[0], gsems[0])}
        h = {0: pltpu.async_copy(he_ref.at[pl.ds(hoff, K)],
                                 hrows.at[0], hsems[0])}
        e = {0: pltpu.async_copy(eye_ref.at[ridx.at[0]], erows.at[0], esems[0])}
        sg, ss0, sq = {}, {}, {}
        for jj in range(BLK1):
            b4 = jj % NBUF
            b2 = jj % 2
            if jj + 1 < BLK1:
                n2 = (jj + 1) % 2
                if jj >= 1:
                    ss0[jj - 1].wait()
                    sq[jj - 1].wait()
                g[jj + 1] = pltpu.async_copy(tab_ref.at[sidx.at[jj + 1]],
                                             rows.at[(jj + 1) % NBUF],
                                             gsems[n2])
                h[jj + 1] = pltpu.async_copy(
                    he_ref.at[pl.ds(hoff + (jj + 1) * K, K)],
                    hrows.at[n2], hsems[n2])
                e[jj + 1] = pltpu.async_copy(eye_ref.at[ridx.at[jj + 1]],
                                             erows.at[n2], esems[n2])
            g[jj].wait()
            sg[jj] = pltpu.async_copy(rows.at[b4], gacc.at[didx.at[jj]],
                                      ssems[b4], add=True)
            h[jj].wait()
            ss0[jj] = pltpu.async_copy(hrows.at[b2], s0acc.at[didx.at[jj]],
                                       s0sems[b2], add=True)
            e[jj].wait()
            sq[jj] = pltpu.async_copy(erows.at[b2], qacc.at[qidx.at[jj]],
                                      qsems[b2], add=True)
        for x in range(BLK1):
            sg[x].wait()
        for x in range(max(0, BLK1 - 2), BLK1):
            ss0[x].wait()
            sq[x].wait()
        return carry

    lax.fori_loop(0, n_chunks // BLK1, block, 0)
    plsc.subcore_barrier()
    pltpu.sync_copy(gacc.at[pl.ds(base, zps)],
                    g_out.at[pl.ds(c * npad + base, zps)])
    pltpu.sync_copy(s0acc.at[pl.ds(base, zps)],
                    s0_out.at[pl.ds(c * npad + base, zps)])

    @pl.when(c == 0)
    def _():
        pltpu.sync_copy(qacc.at[pl.ds(qbase, qps)], deg_out.at[pl.ds(qbase, qps)])


def _sc_spmm_body(tab_ref, src_ref, dst_ref, z16_ref, g_out,
                  gacc, sidx, didx, rows,
                  gsem0, gsem1, ssem0, ssem1, ssem2, ssem3,
                  *, n_chunks, zps, npad):
    c = lax.axis_index("c")
    s = lax.axis_index("s")
    wid = c * NS + s
    base = s * zps
    gsems = (gsem0, gsem1)
    ssems = (ssem0, ssem1, ssem2, ssem3)
    pltpu.sync_copy(z16_ref.at[pl.ds(base, zps)], gacc.at[pl.ds(base, zps)])
    plsc.subcore_barrier()

    def block(blk, carry):
        pltpu.sync_copy(src_ref.at[wid, pl.ds(blk * BLKS, BLKS)], sidx)
        pltpu.sync_copy(dst_ref.at[s, pl.ds(blk * BLKS, BLKS)], didx)
        g = {0: pltpu.async_copy(tab_ref.at[sidx.at[0]], rows.at[0], gsems[0])}
        sg = {}
        for jj in range(BLKS):
            b4 = jj % NBUF
            if jj + 1 < BLKS:
                if jj + 1 >= NBUF:
                    sg[jj + 1 - NBUF].wait()
                g[jj + 1] = pltpu.async_copy(tab_ref.at[sidx.at[jj + 1]],
                                             rows.at[(jj + 1) % NBUF],
                                             gsems[(jj + 1) % 2])
            g[jj].wait()
            sg[jj] = pltpu.async_copy(rows.at[b4], gacc.at[didx.at[jj]],
                                      ssems[b4], add=True)
        for x in range(max(0, BLKS - NBUF), BLKS):
            sg[x].wait()
        return carry

    lax.fori_loop(0, n_chunks // BLKS, block, 0)
    plsc.subcore_barrier()
    pltpu.sync_copy(gacc.at[pl.ds(base, zps)],
                    g_out.at[pl.ds(c * npad + base, zps)])


def kernel(node_feature, edge_feature, edge_index, params):
    N = node_feature.shape[0]
    E = edge_feature.shape[0]
    f32 = jnp.float32

    blk_lcm = NS * K * BLK1 * BLKS // 4
    e_pad = -(-E // blk_lcm) * blk_lcm
    n_chunks = e_pad // (NS * K)
    assert n_chunks % BLK1 == 0 and n_chunks % BLKS == 0
    zps = -(-N // (NS * 128)) * 128
    npad = zps * NS

    we_s, we_d, we_e, be_l, wn_h, wn_a, bn_l = [], [], [], [], [], [], []
    for (We, be, Wn, bn) in params['layers']:
        we_s.append(We[:H]); we_d.append(We[H:2 * H]); we_e.append(We[2 * H:])
        be_l.append(be); wn_h.append(Wn[:H]); wn_a.append(Wn[H:]); bn_l.append(bn)
    P = jnp.eye(H, dtype=f32)
    cvec = jnp.zeros((H,), f32)
    Rs, dls = [], []
    for l in range(3):
        R = P @ we_e[l]
        dl = cvec @ we_e[l] + be_l[l]
        Rs.append(R); dls.append(dl)
        P = P + R
        cvec = cvec + dl

    pad = e_pad - E
    src = jnp.concatenate([edge_index[0], jnp.zeros((pad,), jnp.int32)])
    dst = jnp.concatenate([edge_index[1], jnp.full((pad,), N, jnp.int32)])
    src2 = jnp.reshape(jnp.concatenate([src, src + N]), (NC * NS, n_chunks, K))
    dst3 = jnp.reshape(dst, (NS, n_chunks, K))
    dstq = jnp.reshape(dst // HH, (NS, n_chunks, K))
    dstr = jnp.reshape(dst % HH, (NS, n_chunks, K))
    eye16 = jnp.eye(HH, dtype=f32)
    z16 = jnp.zeros((npad, HH), f32)
    nq = npad // HH
    qps = nq // NS

    (ew1, eb1), (ew2, eb2) = params['edge_mlp']
    eg, ebb = params['ln_edge']
    EB = 4096
    assert e_pad % EB == 0
    ef_pad = jnp.pad(edge_feature, ((0, pad), (0, 0)))
    he0 = _pcall(
        _edge_body,
        grid=(e_pad // EB,),
        in_specs=[pl.BlockSpec((EB, 4), lambda i: (i, 0)),
                  _full(ew1.shape), _full(eb1.shape), _full(ew2.shape),
                  _full(eb2.shape), _full(eg.shape), _full(ebb.shape)],
        out_specs=pl.BlockSpec((NC, EB, HH), lambda i: (0, i, 0)),
        out_shape=jax.ShapeDtypeStruct((NC, e_pad, HH), f32),
    )(ef_pad, ew1, eb1, ew2, eb2, eg, ebb)
    he0_flat = jnp.reshape(he0, (NC * e_pad, HH))

    (nw1, nb1), (nw2, nb2) = params['node_mlp']
    ng, nbb = params['ln_node']
    NB = 2000
    assert N % NB == 0
    ngrid = N // NB
    hn0, a0, t0, b0 = _pcall(
        _node_init_body,
        grid=(ngrid,),
        in_specs=[pl.BlockSpec((NB, 16), lambda i: (i, 0)),
                  _full(nw1.shape), _full(nb1.shape), _full(nw2.shape),
                  _full(nb2.shape), _full(ng.shape), _full(nbb.shape),
                  _full((H, H)), _full((H, H))],
        out_specs=[pl.BlockSpec((NB, H), lambda i: (i, 0)),
                   pl.BlockSpec((NB, H), lambda i: (i, 0)),
                   pl.BlockSpec((NC, NB, HH), lambda i: (0, i, 0)),
                   pl.BlockSpec((NB, H), lambda i: (i, 0))],
        out_shape=[jax.ShapeDtypeStruct((N, H), f32),
                   jax.ShapeDtypeStruct((N, H), f32),
                   jax.ShapeDtypeStruct((NC, N, HH), f32),
                   jax.ShapeDtypeStruct((N, H), f32)],
    )(node_feature, nw1, nb1, nw2, nb2, ng, nbb, we_s[0], we_d[0])

    mesh = plsc.VectorSubcoreMesh(core_axis_name="c", subcore_axis_name="s")
    sc_params = pltpu.CompilerParams(use_tc_tiling_on_sc=False)

    sc_first = pl.kernel(
        functools.partial(_sc_first_body, n_chunks=n_chunks, zps=zps,
                          npad=npad, qps=qps),
        out_type=[jax.ShapeDtypeStruct((NC * npad, HH), f32),
                  jax.ShapeDtypeStruct((nq, HH), f32),
                  jax.ShapeDtypeStruct((NC * npad, HH), f32)],
        mesh=mesh,
        scratch_types=([pltpu.VMEM_SHARED((npad, HH), f32),
                        pltpu.VMEM_SHARED((nq, HH), f32),
                        pltpu.VMEM_SHARED((npad, HH), f32),
                        pltpu.VMEM((BLK1, K), jnp.int32),
                        pltpu.VMEM((BLK1, K), jnp.int32),
                        pltpu.VMEM((BLK1, K), jnp.int32),
                        pltpu.VMEM((BLK1, K), jnp.int32),
                        pltpu.VMEM((NBUF, K, HH), f32),
                        pltpu.VMEM((2, K, HH), f32),
                        pltpu.VMEM((2, K, HH), f32)]
                       + [pltpu.SemaphoreType.DMA] * 14),
        compiler_params=sc_params,
    )
    s0p, deg16, g0p = sc_first(he0_flat, jnp.reshape(t0, (NC * N, HH)),
                               src2, dst3, dstq, dstr, eye16, z16)
    degp = jnp.reshape(deg16, (npad, 1))

    def sc_spmm(tab):
        f = pl.kernel(
            functools.partial(_sc_spmm_body, n_chunks=n_chunks, zps=zps,
                              npad=npad),
            out_type=jax.ShapeDtypeStruct((NC * npad, HH), f32),
            mesh=mesh,
            scratch_types=([pltpu.VMEM_SHARED((npad, HH), f32),
                            pltpu.VMEM((BLKS, K), jnp.int32),
                            pltpu.VMEM((BLKS, K), jnp.int32),
                            pltpu.VMEM((NBUF, K, HH), f32)]
                           + [pltpu.SemaphoreType.DMA] * 6),
            compiler_params=sc_params,
        )
        return f(jnp.reshape(tab, (NC * N, HH)), src2, dst3, z16)

    s0r = jnp.reshape(s0p, (NC, npad, HH))
    g0r = jnp.reshape(g0p, (NC, npad, HH))

    def mid_call(hn, gr, bt, u, v, l):
        return _pcall(
            _mid_body,
            grid=(ngrid,),
            in_specs=[pl.BlockSpec((NB, H), lambda i: (i, 0)),
                      pl.BlockSpec((NC, NB, HH), lambda i: (0, i, 0)),
                      pl.BlockSpec((NC, NB, HH), lambda i: (0, i, 0)),
                      pl.BlockSpec((NB, 1), lambda i: (i, 0)),
                      pl.BlockSpec((NB, H), lambda i: (i, 0)),
                      pl.BlockSpec((NB, H), lambda i: (i, 0)),
                      pl.BlockSpec((NB, H), lambda i: (i, 0)),
                      _full((H, H)), _full((H,)), _full((H, H)),
                      _full((H, H)), _full((H,)),
                      _full((H, H)), _full((H, H)), _full((H, H))],
            out_specs=[pl.BlockSpec((NB, H), lambda i: (i, 0)),
                       pl.BlockSpec((NC, NB, HH), lambda i: (0, i, 0)),
                       pl.BlockSpec((NB, H), lambda i: (i, 0)),
                       pl.BlockSpec((NB, H), lambda i: (i, 0)),
                       pl.BlockSpec((NB, H), lambda i: (i, 0))],
            out_shape=[jax.ShapeDtypeStruct((N, H), f32),
                       jax.ShapeDtypeStruct((NC, N, HH), f32),
                       jax.ShapeDtypeStruct((N, H), f32),
                       jax.ShapeDtypeStruct((N, H), f32),
                       jax.ShapeDtypeStruct((N, H), f32)],
        )(hn, gr, s0r, degp, bt, u, v,
          Rs[l], dls[l], wn_h[l], wn_a[l], bn_l[l],
          we_s[l + 1], we_d[l + 1], we_e[l + 1])

    hn1, t1, bt1, u2, v2 = mid_call(hn0, g0r, b0, a0, b0, 0)
    g1p = sc_spmm(t1)
    hn2, t2, bt2, u3, v3 = mid_call(hn1, jnp.reshape(g1p, (NC, npad, HH)),
                                    bt1, u2, v2, 1)
    g2p = sc_spmm(t2)

    (ow1, ob1), (ow2, ob2) = params['out_mlp']
    out = _pcall(
        _final_body,
        grid=(ngrid,),
        in_specs=[pl.BlockSpec((NB, H), lambda i: (i, 0)),
                  pl.BlockSpec((NC, NB, HH), lambda i: (0, i, 0)),
                  pl.BlockSpec((NC, NB, HH), lambda i: (0, i, 0)),
                  pl.BlockSpec((NB, 1), lambda i: (i, 0)),
                  pl.BlockSpec((NB, H), lambda i: (i, 0)),
                  _full((H, H)), _full((H,)), _full((H, H)),
                  _full((H, H)), _full((H,)),
                  _full(ow1.shape), _full(ob1.shape),
                  _full(ow2.shape), _full(ob2.shape)],
        out_specs=pl.BlockSpec((NB, 3), lambda i: (i, 0)),
        out_shape=jax.ShapeDtypeStruct((N, 3), f32),
    )(hn2, jnp.reshape(g2p, (NC, npad, HH)), s0r, degp, bt2,
      Rs[2], dls[2], wn_h[2], wn_a[2], bn_l[2], ow1, ob1, ow2, ob2)
    return out


# speedup vs baseline: 2.5904x; 2.5904x over previous
import functools

import jax
import jax.numpy as jnp
from jax import lax
from jax.experimental import pallas as pl
from jax.experimental.pallas import tpu as pltpu
from jax.experimental.pallas import tpu_sc as plsc

H = 32
HH = 16
NC = 2
NS = 16
K = 128
BLK1 = 4
BLKS = 8
NBUF = 4

LN_EPS = 1e-5


def _mlp_ln(x, w1_ref, b1_ref, w2_ref, b2_ref, g_ref, bb_ref):
    h = jnp.dot(x, w1_ref[...], preferred_element_type=jnp.float32) + b1_ref[...]
    h = jnp.dot(h, w2_ref[...], preferred_element_type=jnp.float32) + b2_ref[...]
    m = jnp.mean(h, axis=-1, keepdims=True)
    v = jnp.mean((h - m) ** 2, axis=-1, keepdims=True)
    return (h - m) * lax.rsqrt(v + LN_EPS) * g_ref[...] + bb_ref[...]


def _edge_body(x_ref, w1_ref, b1_ref, w2_ref, b2_ref, g_ref, bb_ref, out_ref):
    y = _mlp_ln(x_ref[...], w1_ref, b1_ref, w2_ref, b2_ref, g_ref, bb_ref)
    out_ref[0, :, :] = y[:, :HH]
    out_ref[1, :, :] = y[:, HH:]


def _node_init_body(x_ref, w1_ref, b1_ref, w2_ref, b2_ref, g_ref, bb_ref,
                    wes_ref, wed_ref,
                    hn_ref, a_ref, t_ref, b_ref):
    hn = _mlp_ln(x_ref[...], w1_ref, b1_ref, w2_ref, b2_ref, g_ref, bb_ref)
    a = jnp.dot(hn, wes_ref[...], preferred_element_type=jnp.float32)
    b = jnp.dot(hn, wed_ref[...], preferred_element_type=jnp.float32)
    hn_ref[...] = hn
    a_ref[...] = a
    t_ref[0, :, :] = a[:, :HH]
    t_ref[1, :, :] = a[:, HH:]
    b_ref[...] = b


def _node_update(hn_ref, g_ref, s0_ref, deg_ref, bt_ref, r_ref, dl_ref,
                 wnh_ref, wna_ref, bn_ref):
    hn = hn_ref[...]
    g = jnp.concatenate([g_ref[0], g_ref[1]], axis=-1)
    s0 = jnp.concatenate([s0_ref[0], s0_ref[1]], axis=-1)
    dg = deg_ref[...]
    aggsum = (jnp.dot(s0, r_ref[...], preferred_element_type=jnp.float32)
              + g + dg * bt_ref[...] + dg * dl_ref[...][None, :])
    agg = aggsum / jnp.maximum(dg, 1.0)
    return (hn + jnp.dot(hn, wnh_ref[...], preferred_element_type=jnp.float32)
            + jnp.dot(agg, wna_ref[...], preferred_element_type=jnp.float32)
            + bn_ref[...])


def _mid_body(hn_ref, g_ref, s0_ref, deg_ref, bt_ref, u_ref, v_ref,
              r_ref, dl_ref, wnh_ref, wna_ref, bn_ref,
              wes_ref, wed_ref, wee_ref,
              hno_ref, to_ref, bto_ref, uo_ref, vo_ref):
    hn2 = _node_update(hn_ref, g_ref, s0_ref, deg_ref, bt_ref, r_ref, dl_ref,
                       wnh_ref, wna_ref, bn_ref)
    a = jnp.dot(hn2, wes_ref[...], preferred_element_type=jnp.float32)
    b = jnp.dot(hn2, wed_ref[...], preferred_element_type=jnp.float32)
    at = a + jnp.dot(u_ref[...], wee_ref[...], preferred_element_type=jnp.float32)
    btn = b + jnp.dot(v_ref[...], wee_ref[...], preferred_element_type=jnp.float32)
    hno_ref[...] = hn2
    to_ref[0, :, :] = at[:, :HH]
    to_ref[1, :, :] = at[:, HH:]
    bto_ref[...] = btn
    uo_ref[...] = u_ref[...] + at
    vo_ref[...] = v_ref[...] + btn


def _final_body(hn_ref, g_ref, s0_ref, deg_ref, bt_ref,
                r_ref, dl_ref, wnh_ref, wna_ref, bn_ref,
                wo1_ref, bo1_ref, wo2_ref, bo2_ref, out_ref):
    hn2 = _node_update(hn_ref, g_ref, s0_ref, deg_ref, bt_ref, r_ref, dl_ref,
                       wnh_ref, wna_ref, bn_ref)
    y = jnp.dot(hn2, wo1_ref[...], preferred_element_type=jnp.float32) + bo1_ref[...]
    out_ref[...] = jnp.dot(y, wo2_ref[...], preferred_element_type=jnp.float32) + bo2_ref[...]


def _full(shape):
    nd = len(shape)
    return pl.BlockSpec(shape, lambda i: (0,) * nd)


_pcall = pl.pallas_call


def _sc_first_body(he_ref, tab_ref, src_ref, dst_ref,
                   eye_ref, z16_ref,
                   s0_out, deg_out, g_out,
                   s0acc, qacc, gacc, eyesh, sidx, didx, qidx, ridx, rows,
                   hrows, erows, gsem0, gsem1, ssem0, ssem1, ssem2, ssem3,
                   hsem0, hsem1, s0sem0, s0sem1, esem0, esem1, qsem0, qsem1,
                   *, n_chunks, zps, npad, qps, n_nodes):
    c = lax.axis_index("c")
    s = lax.axis_index("s")
    base = s * zps
    qbase = s * qps
    gsems = (gsem0, gsem1)
    ssems = (ssem0, ssem1, ssem2, ssem3)
    hsems = (hsem0, hsem1)
    s0sems = (s0sem0, s0sem1)
    esems = (esem0, esem1)
    qsems = (qsem0, qsem1)
    pltpu.sync_copy(z16_ref.at[pl.ds(base, zps)], s0acc.at[pl.ds(base, zps)])
    pltpu.sync_copy(z16_ref.at[pl.ds(base, zps)], gacc.at[pl.ds(base, zps)])
    pltpu.sync_copy(z16_ref.at[pl.ds(qbase, qps)], qacc.at[pl.ds(qbase, qps)])

    @pl.when(s == 0)
    def _():
        pltpu.sync_copy(eye_ref, eyesh)

    plsc.subcore_barrier()
    coff = c * n_nodes

    def block(blk, carry):
        pltpu.sync_copy(src_ref.at[s, pl.ds(blk * BLK1, BLK1)], sidx)
        pltpu.sync_copy(dst_ref.at[s, pl.ds(blk * BLK1, BLK1)], didx)
        for jjj in range(BLK1):
            for i in range(K // 16):
                sl = pl.ds(i * 16, 16)
                d16 = didx[jjj, sl]
                sidx[jjj, sl] = sidx[jjj, sl] + coff
                qidx[jjj, sl] = lax.shift_right_logical(d16, 4)
                ridx[jjj, sl] = lax.bitwise_and(d16, 15)
        hoff = c * (NS * n_chunks * K) + s * (n_chunks * K) + blk * (BLK1 * K)
        g = {0: pltpu.async_copy(tab_ref.at[sidx.at[0]], rows.at[0], gsems[0])}
        h = {0: pltpu.async_copy(he_ref.at[pl.ds(hoff, K)],
                                 hrows.at[0], hsems[0])}
        e = {0: pltpu.async_copy(eyesh.at[ridx.at[0]], erows.at[0], esems[0])}
        sg, ss0, sq = {}, {}, {}
        for jj in range(BLK1):
            b4 = jj % NBUF
            b2 = jj % 2
            if jj + 1 < BLK1:
                n2 = (jj + 1) % 2
                if jj >= 1:
                    ss0[jj - 1].wait()
                    sq[jj - 1].wait()
                g[jj + 1] = pltpu.async_copy(tab_ref.at[sidx.at[jj + 1]],
                                             rows.at[(jj + 1) % NBUF],
                                             gsems[n2])
                h[jj + 1] = pltpu.async_copy(
                    he_ref.at[pl.ds(hoff + (jj + 1) * K, K)],
                    hrows.at[n2], hsems[n2])
                e[jj + 1] = pltpu.async_copy(eyesh.at[ridx.at[jj + 1]],
                                             erows.at[n2], esems[n2])
            g[jj].wait()
            sg[jj] = pltpu.async_copy(rows.at[b4], gacc.at[didx.at[jj]],
                                      ssems[b4], add=True)
            h[jj].wait()
            ss0[jj] = pltpu.async_copy(hrows.at[b2], s0acc.at[didx.at[jj]],
                                       s0sems[b2], add=True)
            e[jj].wait()
            sq[jj] = pltpu.async_copy(erows.at[b2], qacc.at[qidx.at[jj]],
                                      qsems[b2], add=True)
        for x in range(BLK1):
            sg[x].wait()
        for x in range(max(0, BLK1 - 2), BLK1):
            ss0[x].wait()
            sq[x].wait()
        return carry

    lax.fori_loop(0, n_chunks // BLK1, block, 0)
    plsc.subcore_barrier()
    pltpu.sync_copy(gacc.at[pl.ds(base, zps)],
                    g_out.at[pl.ds(c * npad + base, zps)])
    pltpu.sync_copy(s0acc.at[pl.ds(base, zps)],
                    s0_out.at[pl.ds(c * npad + base, zps)])

    @pl.when(c == 0)
    def _():
        pltpu.sync_copy(qacc.at[pl.ds(qbase, qps)], deg_out.at[pl.ds(qbase, qps)])


def _sc_spmm_body(tab_ref, src_ref, dst_ref, z16_ref, g_out,
                  gacc, sidx, didx, rows,
                  gsem0, gsem1, ssem0, ssem1, ssem2, ssem3,
                  *, n_chunks, zps, npad, n_nodes):
    c = lax.axis_index("c")
    s = lax.axis_index("s")
    base = s * zps
    gsems = (gsem0, gsem1)
    ssems = (ssem0, ssem1, ssem2, ssem3)
    pltpu.sync_copy(z16_ref.at[pl.ds(base, zps)], gacc.at[pl.ds(base, zps)])
    plsc.subcore_barrier()
    coff = c * n_nodes

    def block(blk, carry):
        pltpu.sync_copy(src_ref.at[s, pl.ds(blk * BLKS, BLKS)], sidx)
        pltpu.sync_copy(dst_ref.at[s, pl.ds(blk * BLKS, BLKS)], didx)
        for jjj in range(BLKS):
            for i in range(K // 16):
                sl = pl.ds(i * 16, 16)
                sidx[jjj, sl] = sidx[jjj, sl] + coff
        g = {0: pltpu.async_copy(tab_ref.at[sidx.at[0]], rows.at[0], gsems[0])}
        sg = {}
        for jj in range(BLKS):
            b4 = jj % NBUF
            if jj + 1 < BLKS:
                if jj + 1 >= NBUF:
                    sg[jj + 1 - NBUF].wait()
                g[jj + 1] = pltpu.async_copy(tab_ref.at[sidx.at[jj + 1]],
                                             rows.at[(jj + 1) % NBUF],
                                             gsems[(jj + 1) % 2])
            g[jj].wait()
            sg[jj] = pltpu.async_copy(rows.at[b4], gacc.at[didx.at[jj]],
                                      ssems[b4], add=True)
        for x in range(max(0, BLKS - NBUF), BLKS):
            sg[x].wait()
        return carry

    lax.fori_loop(0, n_chunks // BLKS, block, 0)
    plsc.subcore_barrier()
    pltpu.sync_copy(gacc.at[pl.ds(base, zps)],
                    g_out.at[pl.ds(c * npad + base, zps)])


def kernel(node_feature, edge_feature, edge_index, params):
    N = node_feature.shape[0]
    E = edge_feature.shape[0]
    f32 = jnp.float32

    blk_lcm = NS * K * BLK1 * BLKS // 4
    e_pad = -(-E // blk_lcm) * blk_lcm
    n_chunks = e_pad // (NS * K)
    assert n_chunks % BLK1 == 0 and n_chunks % BLKS == 0
    zps = -(-N // (NS * 128)) * 128
    npad = zps * NS

    we_s, we_d, we_e, be_l, wn_h, wn_a, bn_l = [], [], [], [], [], [], []
    for (We, be, Wn, bn) in params['layers']:
        we_s.append(We[:H]); we_d.append(We[H:2 * H]); we_e.append(We[2 * H:])
        be_l.append(be); wn_h.append(Wn[:H]); wn_a.append(Wn[H:]); bn_l.append(bn)
    P = jnp.eye(H, dtype=f32)
    cvec = jnp.zeros((H,), f32)
    Rs, dls = [], []
    for l in range(3):
        R = P @ we_e[l]
        dl = cvec @ we_e[l] + be_l[l]
        Rs.append(R); dls.append(dl)
        P = P + R
        cvec = cvec + dl

    pad = e_pad - E
    src = jnp.concatenate([edge_index[0], jnp.zeros((pad,), jnp.int32)])
    dst = jnp.concatenate([edge_index[1], jnp.full((pad,), N, jnp.int32)])
    src3 = jnp.reshape(src, (NS, n_chunks, K))
    dst3 = jnp.reshape(dst, (NS, n_chunks, K))
    eye16 = jnp.eye(HH, dtype=f32)
    z16 = jnp.zeros((npad, HH), f32)
    nq = npad // HH
    qps = nq // NS

    (ew1, eb1), (ew2, eb2) = params['edge_mlp']
    eg, ebb = params['ln_edge']
    EB = 4096
    assert e_pad % EB == 0
    he0 = _pcall(
        _edge_body,
        grid=(e_pad // EB,),
        in_specs=[pl.BlockSpec((EB, 4), lambda i: (i, 0)),
                  _full(ew1.shape), _full(eb1.shape), _full(ew2.shape),
                  _full(eb2.shape), _full(eg.shape), _full(ebb.shape)],
        out_specs=pl.BlockSpec((NC, EB, HH), lambda i: (0, i, 0)),
        out_shape=jax.ShapeDtypeStruct((NC, e_pad, HH), f32),
    )(edge_feature, ew1, eb1, ew2, eb2, eg, ebb)
    he0_flat = jnp.reshape(he0, (NC * e_pad, HH))

    (nw1, nb1), (nw2, nb2) = params['node_mlp']
    ng, nbb = params['ln_node']
    NB = 2000
    assert N % NB == 0
    ngrid = N // NB
    hn0, a0, t0, b0 = _pcall(
        _node_init_body,
        grid=(ngrid,),
        in_specs=[pl.BlockSpec((NB, 16), lambda i: (i, 0)),
                  _full(nw1.shape), _full(nb1.shape), _full(nw2.shape),
                  _full(nb2.shape), _full(ng.shape), _full(nbb.shape),
                  _full((H, H)), _full((H, H))],
        out_specs=[pl.BlockSpec((NB, H), lambda i: (i, 0)),
                   pl.BlockSpec((NB, H), lambda i: (i, 0)),
                   pl.BlockSpec((NC, NB, HH), lambda i: (0, i, 0)),
                   pl.BlockSpec((NB, H), lambda i: (i, 0))],
        out_shape=[jax.ShapeDtypeStruct((N, H), f32),
                   jax.ShapeDtypeStruct((N, H), f32),
                   jax.ShapeDtypeStruct((NC, N, HH), f32),
                   jax.ShapeDtypeStruct((N, H), f32)],
    )(node_feature, nw1, nb1, nw2, nb2, ng, nbb, we_s[0], we_d[0])

    mesh = plsc.VectorSubcoreMesh(core_axis_name="c", subcore_axis_name="s")
    sc_params = pltpu.CompilerParams(use_tc_tiling_on_sc=False)

    sc_first = pl.kernel(
        functools.partial(_sc_first_body, n_chunks=n_chunks, zps=zps,
                          npad=npad, qps=qps, n_nodes=N),
        out_type=[jax.ShapeDtypeStruct((NC * npad, HH), f32),
                  jax.ShapeDtypeStruct((nq, HH), f32),
                  jax.ShapeDtypeStruct((NC * npad, HH), f32)],
        mesh=mesh,
        scratch_types=([pltpu.VMEM_SHARED((npad, HH), f32),
                        pltpu.VMEM_SHARED((nq, HH), f32),
                        pltpu.VMEM_SHARED((npad, HH), f32),
                        pltpu.VMEM_SHARED((HH, HH), f32),
                        pltpu.VMEM((BLK1, K), jnp.int32),
                        pltpu.VMEM((BLK1, K), jnp.int32),
                        pltpu.VMEM((BLK1, K), jnp.int32),
                        pltpu.VMEM((BLK1, K), jnp.int32),
                        pltpu.VMEM((NBUF, K, HH), f32),
                        pltpu.VMEM((2, K, HH), f32),
                        pltpu.VMEM((2, K, HH), f32)]
                       + [pltpu.SemaphoreType.DMA] * 14),
        compiler_params=sc_params,
    )
    s0p, deg16, g0p = sc_first(he0_flat, jnp.reshape(t0, (NC * N, HH)),
                               src3, dst3, eye16, z16)
    degp = jnp.reshape(deg16, (npad, 1))

    def sc_spmm(tab):
        f = pl.kernel(
            functools.partial(_sc_spmm_body, n_chunks=n_chunks, zps=zps,
                              npad=npad, n_nodes=N),
            out_type=jax.ShapeDtypeStruct((NC * npad, HH), f32),
            mesh=mesh,
            scratch_types=([pltpu.VMEM_SHARED((npad, HH), f32),
                            pltpu.VMEM((BLKS, K), jnp.int32),
                            pltpu.VMEM((BLKS, K), jnp.int32),
                            pltpu.VMEM((NBUF, K, HH), f32)]
                           + [pltpu.SemaphoreType.DMA] * 6),
            compiler_params=sc_params,
        )
        return f(jnp.reshape(tab, (NC * N, HH)), src3, dst3, z16)

    s0r = jnp.reshape(s0p, (NC, npad, HH))
    g0r = jnp.reshape(g0p, (NC, npad, HH))

    def mid_call(hn, gr, bt, u, v, l):
        return _pcall(
            _mid_body,
            grid=(ngrid,),
            in_specs=[pl.BlockSpec((NB, H), lambda i: (i, 0)),
                      pl.BlockSpec((NC, NB, HH), lambda i: (0, i, 0)),
                      pl.BlockSpec((NC, NB, HH), lambda i: (0, i, 0)),
                      pl.BlockSpec((NB, 1), lambda i: (i, 0)),
                      pl.BlockSpec((NB, H), lambda i: (i, 0)),
                      pl.BlockSpec((NB, H), lambda i: (i, 0)),
                      pl.BlockSpec((NB, H), lambda i: (i, 0)),
                      _full((H, H)), _full((H,)), _full((H, H)),
                      _full((H, H)), _full((H,)),
                      _full((H, H)), _full((H, H)), _full((H, H))],
            out_specs=[pl.BlockSpec((NB, H), lambda i: (i, 0)),
                       pl.BlockSpec((NC, NB, HH), lambda i: (0, i, 0)),
                       pl.BlockSpec((NB, H), lambda i: (i, 0)),
                       pl.BlockSpec((NB, H), lambda i: (i, 0)),
                       pl.BlockSpec((NB, H), lambda i: (i, 0))],
            out_shape=[jax.ShapeDtypeStruct((N, H), f32),
                       jax.ShapeDtypeStruct((NC, N, HH), f32),
                       jax.ShapeDtypeStruct((N, H), f32),
                       jax.ShapeDtypeStruct((N, H), f32),
                       jax.ShapeDtypeStruct((N, H), f32)],
        )(hn, gr, s0r, degp, bt, u, v,
          Rs[l], dls[l], wn_h[l], wn_a[l], bn_l[l],
          we_s[l + 1], we_d[l + 1], we_e[l + 1])

    hn1, t1, bt1, u2, v2 = mid_call(hn0, g0r, b0, a0, b0, 0)
    g1p = sc_spmm(t1)
    hn2, t2, bt2, u3, v3 = mid_call(hn1, jnp.reshape(g1p, (NC, npad, HH)),
                                    bt1, u2, v2, 1)
    g2p = sc_spmm(t2)

    (ow1, ob1), (ow2, ob2) = params['out_mlp']
    out = _pcall(
        _final_body,
        grid=(ngrid,),
        in_specs=[pl.BlockSpec((NB, H), lambda i: (i, 0)),
                  pl.BlockSpec((NC, NB, HH), lambda i: (0, i, 0)),
                  pl.BlockSpec((NC, NB, HH), lambda i: (0, i, 0)),
                  pl.BlockSpec((NB, 1), lambda i: (i, 0)),
                  pl.BlockSpec((NB, H), lambda i: (i, 0)),
                  _full((H, H)), _full((H,)), _full((H, H)),
                  _full((H, H)), _full((H,)),
                  _full(ow1.shape), _full(ob1.shape),
                  _full(ow2.shape), _full(ob2.shape)],
        out_specs=pl.BlockSpec((NB, 3), lambda i: (i, 0)),
        out_shape=jax.ShapeDtypeStruct((N, 3), f32),
    )(hn2, jnp.reshape(g2p, (NC, npad, HH)), s0r, degp, bt2,
      Rs[2], dls[2], wn_h[2], wn_a[2], bn_l[2], ow1, ob1, ow2, ob2)
    return out


# speedup vs baseline: 2.9076x; 1.1225x over previous
import functools

import jax
import jax.numpy as jnp
from jax import lax
from jax.experimental import pallas as pl
from jax.experimental.pallas import tpu as pltpu
from jax.experimental.pallas import tpu_sc as plsc

H = 32
HH = 16
NC = 2
NS = 16
K = 128
BLK1 = 4
BLKS = 8
NBUF = 4

LN_EPS = 1e-5


def _mlp_ln(x, w1_ref, b1_ref, w2_ref, b2_ref, g_ref, bb_ref):
    h = jnp.dot(x, w1_ref[...], preferred_element_type=jnp.float32) + b1_ref[...]
    h = jnp.dot(h, w2_ref[...], preferred_element_type=jnp.float32) + b2_ref[...]
    m = jnp.mean(h, axis=-1, keepdims=True)
    v = jnp.mean((h - m) ** 2, axis=-1, keepdims=True)
    return (h - m) * lax.rsqrt(v + LN_EPS) * g_ref[...] + bb_ref[...]


def _edge_body(x_ref, w1_ref, b1_ref, w2_ref, b2_ref, g_ref, bb_ref, out_ref):
    y = _mlp_ln(x_ref[...], w1_ref, b1_ref, w2_ref, b2_ref, g_ref, bb_ref)
    eb8 = y.shape[0] // 8
    out_ref[0, :, :] = jnp.concatenate(
        [y[k * eb8:(k + 1) * eb8, :HH] for k in range(8)], axis=1)
    out_ref[1, :, :] = jnp.concatenate(
        [y[k * eb8:(k + 1) * eb8, HH:] for k in range(8)], axis=1)


def _node_init_body(x_ref, w1_ref, b1_ref, w2_ref, b2_ref, g_ref, bb_ref,
                    wes_ref, wed_ref,
                    hn_ref, a_ref, t_ref, b_ref):
    hn = _mlp_ln(x_ref[...], w1_ref, b1_ref, w2_ref, b2_ref, g_ref, bb_ref)
    a = jnp.dot(hn, wes_ref[...], preferred_element_type=jnp.float32)
    b = jnp.dot(hn, wed_ref[...], preferred_element_type=jnp.float32)
    hn_ref[...] = hn
    a_ref[...] = a
    t_ref[0, :, :] = a[:, :HH]
    t_ref[1, :, :] = a[:, HH:]
    b_ref[...] = b


def _node_update(hn_ref, g_ref, s0_ref, deg_ref, bt_ref, r_ref, dl_ref,
                 wnh_ref, wna_ref, bn_ref):
    hn = hn_ref[...]
    g = jnp.concatenate([g_ref[0], g_ref[1]], axis=-1)
    s0 = jnp.concatenate([s0_ref[0], s0_ref[1]], axis=-1)
    dg = deg_ref[...]
    aggsum = (jnp.dot(s0, r_ref[...], preferred_element_type=jnp.float32)
              + g + dg * bt_ref[...] + dg * dl_ref[...][None, :])
    agg = aggsum / jnp.maximum(dg, 1.0)
    return (hn + jnp.dot(hn, wnh_ref[...], preferred_element_type=jnp.float32)
            + jnp.dot(agg, wna_ref[...], preferred_element_type=jnp.float32)
            + bn_ref[...])


def _mid_body(hn_ref, g_ref, s0_ref, deg_ref, bt_ref, u_ref, v_ref,
              r_ref, dl_ref, wnh_ref, wna_ref, bn_ref,
              wes_ref, wed_ref, wee_ref,
              hno_ref, to_ref, bto_ref, uo_ref, vo_ref):
    hn2 = _node_update(hn_ref, g_ref, s0_ref, deg_ref, bt_ref, r_ref, dl_ref,
                       wnh_ref, wna_ref, bn_ref)
    a = jnp.dot(hn2, wes_ref[...], preferred_element_type=jnp.float32)
    b = jnp.dot(hn2, wed_ref[...], preferred_element_type=jnp.float32)
    at = a + jnp.dot(u_ref[...], wee_ref[...], preferred_element_type=jnp.float32)
    btn = b + jnp.dot(v_ref[...], wee_ref[...], preferred_element_type=jnp.float32)
    hno_ref[...] = hn2
    to_ref[0, :, :] = at[:, :HH]
    to_ref[1, :, :] = at[:, HH:]
    bto_ref[...] = btn
    uo_ref[...] = u_ref[...] + at
    vo_ref[...] = v_ref[...] + btn


def _final_body(hn_ref, g_ref, s0_ref, deg_ref, bt_ref,
                r_ref, dl_ref, wnh_ref, wna_ref, bn_ref,
                wo1_ref, bo1_ref, wo2_ref, bo2_ref, out_ref):
    hn2 = _node_update(hn_ref, g_ref, s0_ref, deg_ref, bt_ref, r_ref, dl_ref,
                       wnh_ref, wna_ref, bn_ref)
    y = jnp.dot(hn2, wo1_ref[...], preferred_element_type=jnp.float32) + bo1_ref[...]
    out_ref[...] = jnp.dot(y, wo2_ref[...], preferred_element_type=jnp.float32) + bo2_ref[...]


def _full(shape):
    nd = len(shape)
    return pl.BlockSpec(shape, lambda i: (0,) * nd)


_pcall = pl.pallas_call


def _sc_first_body(he_ref, tab_ref, src_ref, dst_ref,
                   eye_ref, z16_ref,
                   s0_out, deg_out, g_out,
                   s0acc, qacc, gacc, eyesh, sidx, didx, qidx, ridx, rows,
                   hrows, erows, gsem0, gsem1, ssem0, ssem1, ssem2, ssem3,
                   hsem0, hsem1, s0sem0, s0sem1, esem0, esem1, qsem0, qsem1,
                   *, n_chunks, zps, npad, qps, n_nodes):
    c = lax.axis_index("c")
    s = lax.axis_index("s")
    base = s * zps
    qbase = s * qps
    gsems = (gsem0, gsem1)
    ssems = (ssem0, ssem1, ssem2, ssem3)
    hsems = (hsem0, hsem1)
    s0sems = (s0sem0, s0sem1)
    esems = (esem0, esem1)
    qsems = (qsem0, qsem1)
    pltpu.sync_copy(z16_ref.at[pl.ds(base, zps)], s0acc.at[pl.ds(base, zps)])
    pltpu.sync_copy(z16_ref.at[pl.ds(base, zps)], gacc.at[pl.ds(base, zps)])
    pltpu.sync_copy(z16_ref.at[pl.ds(qbase, qps)], qacc.at[pl.ds(qbase, qps)])

    @pl.when(s == 0)
    def _():
        pltpu.sync_copy(eye_ref, eyesh)

    plsc.subcore_barrier()
    coff = c * n_nodes

    def block(blk, carry):
        pltpu.sync_copy(src_ref.at[s, pl.ds(blk * BLK1, BLK1)], sidx)
        pltpu.sync_copy(dst_ref.at[s, pl.ds(blk * BLK1, BLK1)], didx)
        for jjj in range(BLK1):
            for i in range(K // 16):
                sl = pl.ds(i * 16, 16)
                d16 = didx[jjj, sl]
                sidx[jjj, sl] = sidx[jjj, sl] + coff
                qidx[jjj, sl] = lax.shift_right_logical(d16, 4)
                ridx[jjj, sl] = lax.bitwise_and(d16, 15)
        hoff = c * (NS * n_chunks * K) + s * (n_chunks * K) + blk * (BLK1 * K)
        g = {0: pltpu.async_copy(tab_ref.at[sidx.at[0]], rows.at[0], gsems[0])}
        h = {0: pltpu.async_copy(he_ref.at[pl.ds(hoff, K)],
                                 hrows.at[0], hsems[0])}
        e = {0: pltpu.async_copy(eyesh.at[ridx.at[0]], erows.at[0], esems[0])}
        sg, ss0, sq = {}, {}, {}
        for jj in range(BLK1):
            b4 = jj % NBUF
            b2 = jj % 2
            if jj + 1 < BLK1:
                n2 = (jj + 1) % 2
                if jj >= 1:
                    ss0[jj - 1].wait()
                    sq[jj - 1].wait()
                g[jj + 1] = pltpu.async_copy(tab_ref.at[sidx.at[jj + 1]],
                                             rows.at[(jj + 1) % NBUF],
                                             gsems[n2])
                h[jj + 1] = pltpu.async_copy(
                    he_ref.at[pl.ds(hoff + (jj + 1) * K, K)],
                    hrows.at[n2], hsems[n2])
                e[jj + 1] = pltpu.async_copy(eyesh.at[ridx.at[jj + 1]],
                                             erows.at[n2], esems[n2])
            g[jj].wait()
            sg[jj] = pltpu.async_copy(rows.at[b4], gacc.at[didx.at[jj]],
                                      ssems[b4], add=True)
            h[jj].wait()
            ss0[jj] = pltpu.async_copy(hrows.at[b2], s0acc.at[didx.at[jj]],
                                       s0sems[b2], add=True)
            e[jj].wait()
            sq[jj] = pltpu.async_copy(erows.at[b2], qacc.at[qidx.at[jj]],
                                      qsems[b2], add=True)
        for x in range(BLK1):
            sg[x].wait()
        for x in range(max(0, BLK1 - 2), BLK1):
            ss0[x].wait()
            sq[x].wait()
        return carry

    lax.fori_loop(0, n_chunks // BLK1, block, 0)
    plsc.subcore_barrier()
    pltpu.sync_copy(gacc.at[pl.ds(base, zps)],
                    g_out.at[pl.ds(c * npad + base, zps)])
    pltpu.sync_copy(s0acc.at[pl.ds(base, zps)],
                    s0_out.at[pl.ds(c * npad + base, zps)])

    @pl.when(c == 0)
    def _():
        pltpu.sync_copy(qacc.at[pl.ds(qbase, qps)], deg_out.at[pl.ds(qbase, qps)])


def _sc_spmm_body(tab_ref, src_ref, dst_ref, z16_ref, g_out,
                  gacc, sidx, didx, rows,
                  gsem0, gsem1, ssem0, ssem1, ssem2, ssem3,
                  *, n_chunks, zps, npad, n_nodes):
    c = lax.axis_index("c")
    s = lax.axis_index("s")
    base = s * zps
    gsems = (gsem0, gsem1)
    ssems = (ssem0, ssem1, ssem2, ssem3)
    pltpu.sync_copy(z16_ref.at[pl.ds(base, zps)], gacc.at[pl.ds(base, zps)])
    plsc.subcore_barrier()
    coff = c * n_nodes

    def block(blk, carry):
        pltpu.sync_copy(src_ref.at[s, pl.ds(blk * BLKS, BLKS)], sidx)
        pltpu.sync_copy(dst_ref.at[s, pl.ds(blk * BLKS, BLKS)], didx)
        for jjj in range(BLKS):
            for i in range(K // 16):
                sl = pl.ds(i * 16, 16)
                sidx[jjj, sl] = sidx[jjj, sl] + coff
        g = {0: pltpu.async_copy(tab_ref.at[sidx.at[0]], rows.at[0], gsems[0])}
        sg = {}
        for jj in range(BLKS):
            b4 = jj % NBUF
            if jj + 1 < BLKS:
                if jj + 1 >= NBUF:
                    sg[jj + 1 - NBUF].wait()
                g[jj + 1] = pltpu.async_copy(tab_ref.at[sidx.at[jj + 1]],
                                             rows.at[(jj + 1) % NBUF],
                                             gsems[(jj + 1) % 2])
            g[jj].wait()
            sg[jj] = pltpu.async_copy(rows.at[b4], gacc.at[didx.at[jj]],
                                      ssems[b4], add=True)
        for x in range(max(0, BLKS - NBUF), BLKS):
            sg[x].wait()
        return carry

    lax.fori_loop(0, n_chunks // BLKS, block, 0)
    plsc.subcore_barrier()
    pltpu.sync_copy(gacc.at[pl.ds(base, zps)],
                    g_out.at[pl.ds(c * npad + base, zps)])


def kernel(node_feature, edge_feature, edge_index, params):
    N = node_feature.shape[0]
    E = edge_feature.shape[0]
    f32 = jnp.float32

    blk_lcm = NS * K * BLK1 * BLKS // 4
    e_pad = -(-E // blk_lcm) * blk_lcm
    n_chunks = e_pad // (NS * K)
    assert n_chunks % BLK1 == 0 and n_chunks % BLKS == 0
    zps = -(-N // (NS * 128)) * 128
    npad = zps * NS

    we_s, we_d, we_e, be_l, wn_h, wn_a, bn_l = [], [], [], [], [], [], []
    for (We, be, Wn, bn) in params['layers']:
        we_s.append(We[:H]); we_d.append(We[H:2 * H]); we_e.append(We[2 * H:])
        be_l.append(be); wn_h.append(Wn[:H]); wn_a.append(Wn[H:]); bn_l.append(bn)
    P = jnp.eye(H, dtype=f32)
    cvec = jnp.zeros((H,), f32)
    Rs, dls = [], []
    for l in range(3):
        R = P @ we_e[l]
        dl = cvec @ we_e[l] + be_l[l]
        Rs.append(R); dls.append(dl)
        P = P + R
        cvec = cvec + dl

    EB = 4096
    pad = e_pad - E
    src = jnp.concatenate([edge_index[0], jnp.zeros((pad,), jnp.int32)])
    dst = jnp.concatenate([edge_index[1], jnp.full((pad,), N, jnp.int32)])
    def _perm(a):
        a = jnp.reshape(a, (e_pad // EB, 8, EB // 8))
        return jnp.reshape(jnp.transpose(a, (0, 2, 1)), (NS, n_chunks, K))
    src3 = _perm(src)
    dst3 = _perm(dst)
    eye16 = jnp.eye(HH, dtype=f32)
    z16 = jnp.zeros((npad, HH), f32)
    nq = npad // HH
    qps = nq // NS

    (ew1, eb1), (ew2, eb2) = params['edge_mlp']
    eg, ebb = params['ln_edge']
    assert e_pad % EB == 0
    he0 = _pcall(
        _edge_body,
        grid=(e_pad // EB,),
        in_specs=[pl.BlockSpec((EB, 4), lambda i: (i, 0)),
                  _full(ew1.shape), _full(eb1.shape), _full(ew2.shape),
                  _full(eb2.shape), _full(eg.shape), _full(ebb.shape)],
        out_specs=pl.BlockSpec((NC, EB // 8, 128), lambda i: (0, i, 0)),
        out_shape=jax.ShapeDtypeStruct((NC, e_pad // 8, 128), f32),
    )(edge_feature, ew1, eb1, ew2, eb2, eg, ebb)
    he0_flat = jnp.reshape(he0, (NC * e_pad, HH))

    (nw1, nb1), (nw2, nb2) = params['node_mlp']
    ng, nbb = params['ln_node']
    NB = 1024
    assert npad % NB == 0
    ngrid = npad // NB
    hn0, a0, t0, b0 = _pcall(
        _node_init_body,
        grid=(ngrid,),
        in_specs=[pl.BlockSpec((NB, 16), lambda i: (i, 0)),
                  _full(nw1.shape), _full(nb1.shape), _full(nw2.shape),
                  _full(nb2.shape), _full(ng.shape), _full(nbb.shape),
                  _full((H, H)), _full((H, H))],
        out_specs=[pl.BlockSpec((NB, H), lambda i: (i, 0)),
                   pl.BlockSpec((NB, H), lambda i: (i, 0)),
                   pl.BlockSpec((NC, NB, HH), lambda i: (0, i, 0)),
                   pl.BlockSpec((NB, H), lambda i: (i, 0))],
        out_shape=[jax.ShapeDtypeStruct((npad, H), f32),
                   jax.ShapeDtypeStruct((npad, H), f32),
                   jax.ShapeDtypeStruct((NC, npad, HH), f32),
                   jax.ShapeDtypeStruct((npad, H), f32)],
    )(jnp.pad(node_feature, ((0, npad - N), (0, 0))),
      nw1, nb1, nw2, nb2, ng, nbb, we_s[0], we_d[0])

    mesh = plsc.VectorSubcoreMesh(core_axis_name="c", subcore_axis_name="s")
    sc_params = pltpu.CompilerParams(use_tc_tiling_on_sc=False)

    sc_first = pl.kernel(
        functools.partial(_sc_first_body, n_chunks=n_chunks, zps=zps,
                          npad=npad, qps=qps, n_nodes=npad),
        out_type=[jax.ShapeDtypeStruct((NC * npad, HH), f32),
                  jax.ShapeDtypeStruct((nq, HH), f32),
                  jax.ShapeDtypeStruct((NC * npad, HH), f32)],
        mesh=mesh,
        scratch_types=([pltpu.VMEM_SHARED((npad, HH), f32),
                        pltpu.VMEM_SHARED((nq, HH), f32),
                        pltpu.VMEM_SHARED((npad, HH), f32),
                        pltpu.VMEM_SHARED((HH, HH), f32),
                        pltpu.VMEM((BLK1, K), jnp.int32),
                        pltpu.VMEM((BLK1, K), jnp.int32),
                        pltpu.VMEM((BLK1, K), jnp.int32),
                        pltpu.VMEM((BLK1, K), jnp.int32),
                        pltpu.VMEM((NBUF, K, HH), f32),
                        pltpu.VMEM((2, K, HH), f32),
                        pltpu.VMEM((2, K, HH), f32)]
                       + [pltpu.SemaphoreType.DMA] * 14),
        compiler_params=sc_params,
    )
    s0p, deg16, g0p = sc_first(he0_flat, jnp.reshape(t0, (NC * npad, HH)),
                               src3, dst3, eye16, z16)
    degp = jnp.reshape(deg16, (npad, 1))

    def sc_spmm(tab):
        f = pl.kernel(
            functools.partial(_sc_spmm_body, n_chunks=n_chunks, zps=zps,
                              npad=npad, n_nodes=npad),
            out_type=jax.ShapeDtypeStruct((NC * npad, HH), f32),
            mesh=mesh,
            scratch_types=([pltpu.VMEM_SHARED((npad, HH), f32),
                            pltpu.VMEM((BLKS, K), jnp.int32),
                            pltpu.VMEM((BLKS, K), jnp.int32),
                            pltpu.VMEM((NBUF, K, HH), f32)]
                           + [pltpu.SemaphoreType.DMA] * 6),
            compiler_params=sc_params,
        )
        return f(jnp.reshape(tab, (NC * npad, HH)), src3, dst3, z16)

    s0r = jnp.reshape(s0p, (NC, npad, HH))
    g0r = jnp.reshape(g0p, (NC, npad, HH))

    def mid_call(hn, gr, bt, u, v, l):
        return _pcall(
            _mid_body,
            grid=(ngrid,),
            in_specs=[pl.BlockSpec((NB, H), lambda i: (i, 0)),
                      pl.BlockSpec((NC, NB, HH), lambda i: (0, i, 0)),
                      pl.BlockSpec((NC, NB, HH), lambda i: (0, i, 0)),
                      pl.BlockSpec((NB, 1), lambda i: (i, 0)),
                      pl.BlockSpec((NB, H), lambda i: (i, 0)),
                      pl.BlockSpec((NB, H), lambda i: (i, 0)),
                      pl.BlockSpec((NB, H), lambda i: (i, 0)),
                      _full((H, H)), _full((H,)), _full((H, H)),
                      _full((H, H)), _full((H,)),
                      _full((H, H)), _full((H, H)), _full((H, H))],
            out_specs=[pl.BlockSpec((NB, H), lambda i: (i, 0)),
                       pl.BlockSpec((NC, NB, HH), lambda i: (0, i, 0)),
                       pl.BlockSpec((NB, H), lambda i: (i, 0)),
                       pl.BlockSpec((NB, H), lambda i: (i, 0)),
                       pl.BlockSpec((NB, H), lambda i: (i, 0))],
            out_shape=[jax.ShapeDtypeStruct((npad, H), f32),
                       jax.ShapeDtypeStruct((NC, npad, HH), f32),
                       jax.ShapeDtypeStruct((npad, H), f32),
                       jax.ShapeDtypeStruct((npad, H), f32),
                       jax.ShapeDtypeStruct((npad, H), f32)],
        )(hn, gr, s0r, degp, bt, u, v,
          Rs[l], dls[l], wn_h[l], wn_a[l], bn_l[l],
          we_s[l + 1], we_d[l + 1], we_e[l + 1])

    hn1, t1, bt1, u2, v2 = mid_call(hn0, g0r, b0, a0, b0, 0)
    g1p = sc_spmm(t1)
    hn2, t2, bt2, u3, v3 = mid_call(hn1, jnp.reshape(g1p, (NC, npad, HH)),
                                    bt1, u2, v2, 1)
    g2p = sc_spmm(t2)

    (ow1, ob1), (ow2, ob2) = params['out_mlp']
    out = _pcall(
        _final_body,
        grid=(ngrid,),
        in_specs=[pl.BlockSpec((NB, H), lambda i: (i, 0)),
                  pl.BlockSpec((NC, NB, HH), lambda i: (0, i, 0)),
                  pl.BlockSpec((NC, NB, HH), lambda i: (0, i, 0)),
                  pl.BlockSpec((NB, 1), lambda i: (i, 0)),
                  pl.BlockSpec((NB, H), lambda i: (i, 0)),
                  _full((H, H)), _full((H,)), _full((H, H)),
                  _full((H, H)), _full((H,)),
                  _full(ow1.shape), _full(ob1.shape),
                  _full(ow2.shape), _full(ob2.shape)],
        out_specs=pl.BlockSpec((NB, 3), lambda i: (i, 0)),
        out_shape=jax.ShapeDtypeStruct((npad, 3), f32),
    )(hn2, jnp.reshape(g2p, (NC, npad, HH)), s0r, degp, bt2,
      Rs[2], dls[2], wn_h[2], wn_a[2], bn_l[2], ow1, ob1, ow2, ob2)
    return out[:N]
